# baseline (speedup 1.0000x reference)
import functools

import jax
import jax.numpy as jnp
from jax.experimental import pallas as pl
from jax.experimental.pallas import tpu as pltpu
from jax.experimental.pallas import tpu_sc as plsc

_N = 10000
_E = 320000
_G = 64
_IN = 128
_HID = 256
_CK = 64
_NC = 2
_NS = 16
_B = 100
_NB = _E // _NS // _B
_DB = 100
_NBUF = 4
_RB = 10
_R = _N // _RB
_RPS = 632
_LAST = _N - (_NS - 1) * _RPS


def _sub_rows(s, do):
    @pl.when(s < _NS - 1)
    def _():
        do(s * _RPS, _RPS)

    @pl.when(s == _NS - 1)
    def _():
        do((_NS - 1) * _RPS, _LAST)


@functools.cache
def _sc_mesh():
    return plsc.VectorSubcoreMesh(core_axis_name="c", subcore_axis_name="s",
                                  num_cores=_NC, num_subcores=_NS)


def _deg_body(dst3_hbm, zeros_hbm, out0, out1, stage_v, ones_v, acc_sh, sem):
    c = jax.lax.axis_index("c")
    s = jax.lax.axis_index("s")

    ones = jnp.ones((16,), jnp.float32)

    @pl.loop(0, _B)
    def _(i):
        ones_v[i, pl.ds(0, 16)] = ones

    _sub_rows(s, lambda off, n: pltpu.sync_copy(
        zeros_hbm.at[pl.ds(0, n)], acc_sh.at[pl.ds(off, n)]))
    pltpu.sync_copy(dst3_hbm.at[s, pl.ds(c * _DB, _DB)], stage_v)
    plsc.subcore_barrier()

    @pl.loop(0, _DB // 5)
    def _(g):
        for j in range(5):
            pltpu.async_copy(ones_v, acc_sh.at[stage_v.at[5 * g + j]],
                             sem, add=True)
        for j in range(5):
            pltpu.make_async_copy(ones_v, acc_sh.at[stage_v.at[0]],
                                  sem).wait()

    plsc.subcore_barrier()

    def drain(out_ref):
        _sub_rows(s, lambda off, n: pltpu.sync_copy(
            acc_sh.at[pl.ds(off, n)], out_ref.at[pl.ds(off, n)]))

    @pl.when(c == 0)
    def _():
        drain(out0)

    @pl.when(c == 1)
    def _():
        drain(out1)


@functools.cache
def _deg_kernel():
    return pl.kernel(
        _deg_body,
        out_type=[jax.ShapeDtypeStruct((_N, 16), jnp.float32)] * 2,
        mesh=_sc_mesh(),
        scratch_types=[
            pltpu.VMEM((_DB, _B), jnp.int32),
            pltpu.VMEM((_B, 16), jnp.float32),
            pltpu.VMEM_SHARED((_N, 16), jnp.float32),
            pltpu.SemaphoreType.DMA,
        ],
        compiler_params=pltpu.CompilerParams(use_tc_tiling_on_sc=False),
    )


def _layer_body(y0_hbm, y1_hbm, y2_hbm, y3_hbm, src3_hbm, dst3_hbm,
                zeros_hbm, out0, out1, out2, out3,
                sstage, dstage, gbufs, acc_sh, gsems, ssems):
    c = jax.lax.axis_index("c")
    s = jax.lax.axis_index("s")

    def zero_acc():
        _sub_rows(s, lambda off, n: pltpu.sync_copy(
            zeros_hbm.at[pl.ds(0, n)], acc_sh.at[pl.ds(off, n)]))

    zero_acc()
    pltpu.sync_copy(src3_hbm.at[s], sstage)
    pltpu.sync_copy(dst3_hbm.at[s], dstage)
    plsc.subcore_barrier()

    def accumulate(y_ref, out_ref, rezero):
        def g_start(i, j):
            pltpu.async_copy(y_ref.at[sstage.at[i]], gbufs.at[j], gsems.at[j])

        def g_wait(j):
            pltpu.make_async_copy(y_ref.at[sstage.at[0]], gbufs.at[j],
                                  gsems.at[j]).wait()

        def s_start(i, j):
            pltpu.async_copy(gbufs.at[j], acc_sh.at[dstage.at[i]],
                             ssems.at[j], add=True)

        def s_wait(j):
            pltpu.make_async_copy(gbufs.at[j], acc_sh.at[dstage.at[0]],
                                  ssems.at[j]).wait()

        for j in range(_NBUF):
            g_start(j, j)

        @pl.loop(0, _NB // _NBUF)
        def _(k):
            i = _NBUF * k
            for j in range(_NBUF):
                g_wait(j)
                s_start(i + j, j)

            @pl.when(k < _NB // _NBUF - 1)
            def _():
                for j in range(_NBUF):
                    s_wait(j)
                    g_start(i + _NBUF + j, j)

        for j in range(_NBUF):
            s_wait(j)
        plsc.subcore_barrier()
        _sub_rows(s, lambda off, n: pltpu.sync_copy(
            acc_sh.at[pl.ds(off, n)], out_ref.at[pl.ds(off, n)]))
        if rezero:
            plsc.subcore_barrier()
            zero_acc()
            plsc.subcore_barrier()

    @pl.when(c == 0)
    def _():
        accumulate(y0_hbm, out0, True)
        accumulate(y1_hbm, out1, False)

    @pl.when(c == 1)
    def _():
        accumulate(y2_hbm, out2, True)
        accumulate(y3_hbm, out3, False)


@functools.cache
def _layer_kernel():
    return pl.kernel(
        _layer_body,
        out_type=[jax.ShapeDtypeStruct((_N, _CK), jnp.float32)] * 4,
        mesh=_sc_mesh(),
        scratch_types=[
            pltpu.VMEM((_NB, _B), jnp.int32),
            pltpu.VMEM((_NB, _B), jnp.int32),
            pltpu.VMEM((_NBUF, _B, _CK), jnp.float32),
            pltpu.VMEM_SHARED((_N, _CK), jnp.float32),
            pltpu.SemaphoreType.DMA((_NBUF,)),
            pltpu.SemaphoreType.DMA((_NBUF,)),
        ],
        compiler_params=pltpu.CompilerParams(use_tc_tiling_on_sc=False),
    )


def _chunk_specs(n):
    return [pl.BlockSpec((_R, _CK), lambda i: (i, 0))] * n


def _dinv_of(deg0_ref, deg1_ref):
    return jax.lax.rsqrt(deg0_ref[:, 0] + deg1_ref[:, 0] + 1.0)


def _tc1a_body(x_ref, w1_ref, *xw_refs):
    xw = jnp.dot(x_ref[...], w1_ref[...], preferred_element_type=jnp.float32)
    for k in range(4):
        xw_refs[k][...] = xw[:, k * _CK:(k + 1) * _CK]


def _tc1a(x, w1):
    return pl.pallas_call(
        _tc1a_body,
        grid=(_RB,),
        in_specs=[
            pl.BlockSpec((_R, _IN), lambda i: (i, 0)),
            pl.BlockSpec((_IN, _HID), lambda i: (0, 0)),
        ],
        out_specs=_chunk_specs(4),
        out_shape=[jax.ShapeDtypeStruct((_N, _CK), jnp.float32)] * 4,
    )(x, w1)


def _tc1b_body(deg0_ref, deg1_ref, *refs):
    xw_refs, y_refs = refs[0:4], refs[4:8]
    dinv = _dinv_of(deg0_ref, deg1_ref)
    for k in range(4):
        y_refs[k][...] = dinv[:, None] * xw_refs[k][...]


def _tc1b(deg0, deg1, xws):
    return pl.pallas_call(
        _tc1b_body,
        grid=(_RB,),
        in_specs=[
            pl.BlockSpec((_R, 16), lambda i: (i, 0)),
            pl.BlockSpec((_R, 16), lambda i: (i, 0)),
            *_chunk_specs(4),
        ],
        out_specs=_chunk_specs(4),
        out_shape=[jax.ShapeDtypeStruct((_N, _CK), jnp.float32)] * 4,
    )(deg0, deg1, *xws)


def _sum_rows(a_refs, y_refs):
    return jnp.concatenate(
        [a_refs[k][...] + y_refs[k][...] for k in range(4)], axis=1)


def _tc2_body(deg0_ref, deg1_ref, *refs):
    a_refs, y_refs = refs[0:4], refs[4:8]
    b1_ref, w2_ref = refs[8], refs[9]
    o_refs = refs[10:14]
    dinv = _dinv_of(deg0_ref, deg1_ref)
    h = jnp.maximum(dinv[:, None] * _sum_rows(a_refs, y_refs) + b1_ref[...],
                    0.0)
    y2 = dinv[:, None] * jnp.dot(h, w2_ref[...],
                                 preferred_element_type=jnp.float32)
    for k in range(4):
        o_refs[k][...] = y2[:, k * _CK:(k + 1) * _CK]


def _tc2(deg0, deg1, accs, ys, b1, w2):
    return pl.pallas_call(
        _tc2_body,
        grid=(_RB,),
        in_specs=[
            pl.BlockSpec((_R, 16), lambda i: (i, 0)),
            pl.BlockSpec((_R, 16), lambda i: (i, 0)),
            *_chunk_specs(8),
            pl.BlockSpec((1, _HID), lambda i: (0, 0)),
            pl.BlockSpec((_HID, _HID), lambda i: (0, 0)),
        ],
        out_specs=_chunk_specs(4),
        out_shape=[jax.ShapeDtypeStruct((_N, _CK), jnp.float32)] * 4,
    )(deg0, deg1, *accs, *ys, b1, w2)


def _tc3_body(deg0_ref, deg1_ref, *refs):
    a_refs, y_refs = refs[0:4], refs[4:8]
    b2_ref, batch_ref, wl_ref, bl_ref = refs[8:12]
    out_ref = refs[12]
    pool_acc, cnt_acc = refs[13], refs[14]
    i = pl.program_id(0)
    dinv = _dinv_of(deg0_ref, deg1_ref)
    h = jnp.maximum(dinv[:, None] * _sum_rows(a_refs, y_refs) + b2_ref[...],
                    0.0)
    b = batch_ref[0, 0, :]
    seg = jax.lax.broadcasted_iota(jnp.int32, (_G, _R), 0)
    p = (b[None, :] == seg).astype(jnp.float32)
    ppart = jnp.dot(p, h, preferred_element_type=jnp.float32)
    cpart = jnp.sum(p, axis=1, keepdims=True)

    @pl.when(i == 0)
    def _():
        pool_acc[...] = ppart
        cnt_acc[...] = cpart

    @pl.when(i > 0)
    def _():
        pool_acc[...] += ppart
        cnt_acc[...] += cpart

    @pl.when(i == _RB - 1)
    def _():
        pooled = pool_acc[...] / jnp.maximum(cnt_acc[...], 1.0)
        out_ref[...] = jnp.dot(pooled, wl_ref[...],
                               preferred_element_type=jnp.float32) + bl_ref[...]


def _tc3(deg0, deg1, accs, ys, b2, batch3, wl, bl):
    return pl.pallas_call(
        _tc3_body,
        grid=(_RB,),
        in_specs=[
            pl.BlockSpec((_R, 16), lambda i: (i, 0)),
            pl.BlockSpec((_R, 16), lambda i: (i, 0)),
            *_chunk_specs(8),
            pl.BlockSpec((1, _HID), lambda i: (0, 0)),
            pl.BlockSpec((1, 1, _R), lambda i: (i, 0, 0)),
            pl.BlockSpec((_HID, 1), lambda i: (0, 0)),
            pl.BlockSpec((1, 1), lambda i: (0, 0)),
        ],
        out_specs=pl.BlockSpec((_G, 1), lambda i: (0, 0)),
        out_shape=jax.ShapeDtypeStruct((_G, 1), jnp.float32),
        scratch_shapes=[
            pltpu.VMEM((_G, _HID), jnp.float32),
            pltpu.VMEM((_G, 1), jnp.float32),
        ],
    )(deg0, deg1, *accs, *ys, b2, batch3, wl, bl)


def _deg_call(dst3, zeros):
    return _deg_kernel()(dst3, zeros)


def _layer_call(ys, src3, dst3, zeros):
    return _layer_kernel()(*ys, src3, dst3, zeros)


def kernel(x, edge_index, batch, W1, b1, W2, b2, Wl, bl):
    src = edge_index[0]
    dst = edge_index[1]
    src3 = src.reshape(_NS, _NB, _B)
    dst3 = dst.reshape(_NS, _NB, _B)
    zeros = jnp.zeros((_RPS, _CK), jnp.float32)
    zeros16 = jnp.zeros((_RPS, 16), jnp.float32)
    deg0, deg1 = _deg_call(dst3, zeros16)
    xws = _tc1a(x, W1)
    ys = _tc1b(deg0, deg1, xws)
    accs = _layer_call(ys, src3, dst3, zeros)
    zs = _tc2(deg0, deg1, accs, ys, b1.reshape(1, _HID), W2)
    accs2 = _layer_call(zs, src3, dst3, zeros)
    return _tc3(deg0, deg1, accs2, zs, b2.reshape(1, _HID),
                batch.reshape(_RB, 1, _R), Wl, bl.reshape(1, 1))


# speedup vs baseline: 1.0245x; 1.0245x over previous
import functools

import jax
import jax.numpy as jnp
from jax.experimental import pallas as pl
from jax.experimental.pallas import tpu as pltpu
from jax.experimental.pallas import tpu_sc as plsc

_N = 10000
_E = 320000
_G = 64
_IN = 128
_HID = 256
_CK = 64
_NC = 2
_NS = 16
_B = 125
_NB = _E // _NS // _B
_DB = _NB // _NC
_NBUF = 4
_RB = 10
_R = _N // _RB
_RPS = 632
_LAST = _N - (_NS - 1) * _RPS


def _sub_rows(s, do):
    @pl.when(s < _NS - 1)
    def _():
        do(s * _RPS, _RPS)

    @pl.when(s == _NS - 1)
    def _():
        do((_NS - 1) * _RPS, _LAST)


@functools.cache
def _sc_mesh():
    return plsc.VectorSubcoreMesh(core_axis_name="c", subcore_axis_name="s",
                                  num_cores=_NC, num_subcores=_NS)


def _deg_body(dst3_hbm, zeros_hbm, out0, out1, stage_v, ones_v, acc_sh, sem):
    c = jax.lax.axis_index("c")
    s = jax.lax.axis_index("s")

    ones = jnp.ones((16,), jnp.float32)

    @pl.loop(0, _B)
    def _(i):
        ones_v[i, pl.ds(0, 16)] = ones

    _sub_rows(s, lambda off, n: pltpu.sync_copy(
        zeros_hbm.at[pl.ds(0, n)], acc_sh.at[pl.ds(off, n)]))
    pltpu.sync_copy(dst3_hbm.at[s, pl.ds(c * _DB, _DB)], stage_v)
    plsc.subcore_barrier()

    @pl.loop(0, _DB // 5)
    def _(g):
        for j in range(5):
            pltpu.async_copy(ones_v, acc_sh.at[stage_v.at[5 * g + j]],
                             sem, add=True)
        for j in range(5):
            pltpu.make_async_copy(ones_v, acc_sh.at[stage_v.at[0]],
                                  sem).wait()

    plsc.subcore_barrier()

    def drain(out_ref):
        _sub_rows(s, lambda off, n: pltpu.sync_copy(
            acc_sh.at[pl.ds(off, n)], out_ref.at[pl.ds(off, n)]))

    @pl.when(c == 0)
    def _():
        drain(out0)

    @pl.when(c == 1)
    def _():
        drain(out1)


@functools.cache
def _deg_kernel():
    return pl.kernel(
        _deg_body,
        out_type=[jax.ShapeDtypeStruct((_N, 16), jnp.float32)] * 2,
        mesh=_sc_mesh(),
        scratch_types=[
            pltpu.VMEM((_DB, _B), jnp.int32),
            pltpu.VMEM((_B, 16), jnp.float32),
            pltpu.VMEM_SHARED((_N, 16), jnp.float32),
            pltpu.SemaphoreType.DMA,
        ],
        compiler_params=pltpu.CompilerParams(use_tc_tiling_on_sc=False),
    )


def _layer_body(y0_hbm, y1_hbm, y2_hbm, y3_hbm, src3_hbm, dst3_hbm,
                zeros_hbm, out0, out1, out2, out3,
                sstage, dstage, gbufs, acc_sh, gsems, ssems):
    c = jax.lax.axis_index("c")
    s = jax.lax.axis_index("s")

    def zero_acc():
        _sub_rows(s, lambda off, n: pltpu.sync_copy(
            zeros_hbm.at[pl.ds(0, n)], acc_sh.at[pl.ds(off, n)]))

    zero_acc()
    pltpu.sync_copy(src3_hbm.at[s], sstage)
    pltpu.sync_copy(dst3_hbm.at[s], dstage)
    plsc.subcore_barrier()

    def accumulate(y_ref, out_ref, rezero):
        def g_start(i, j):
            pltpu.async_copy(y_ref.at[sstage.at[i]], gbufs.at[j], gsems.at[j])

        def g_wait(j):
            pltpu.make_async_copy(y_ref.at[sstage.at[0]], gbufs.at[j],
                                  gsems.at[j]).wait()

        def s_start(i, j):
            pltpu.async_copy(gbufs.at[j], acc_sh.at[dstage.at[i]],
                             ssems.at[j], add=True)

        def s_wait(j):
            pltpu.make_async_copy(gbufs.at[j], acc_sh.at[dstage.at[0]],
                                  ssems.at[j]).wait()

        for j in range(_NBUF):
            g_start(j, j)

        @pl.loop(0, _NB // _NBUF)
        def _(k):
            i = _NBUF * k
            for j in range(_NBUF):
                g_wait(j)
                s_start(i + j, j)

            @pl.when(k < _NB // _NBUF - 1)
            def _():
                for j in range(_NBUF):
                    s_wait(j)
                    g_start(i + _NBUF + j, j)

        for j in range(_NBUF):
            s_wait(j)
        plsc.subcore_barrier()
        _sub_rows(s, lambda off, n: pltpu.sync_copy(
            acc_sh.at[pl.ds(off, n)], out_ref.at[pl.ds(off, n)]))
        if rezero:
            plsc.subcore_barrier()
            zero_acc()
            plsc.subcore_barrier()

    @pl.when(c == 0)
    def _():
        accumulate(y0_hbm, out0, True)
        accumulate(y1_hbm, out1, False)

    @pl.when(c == 1)
    def _():
        accumulate(y2_hbm, out2, True)
        accumulate(y3_hbm, out3, False)


@functools.cache
def _layer_kernel():
    return pl.kernel(
        _layer_body,
        out_type=[jax.ShapeDtypeStruct((_N, _CK), jnp.float32)] * 4,
        mesh=_sc_mesh(),
        scratch_types=[
            pltpu.VMEM((_NB, _B), jnp.int32),
            pltpu.VMEM((_NB, _B), jnp.int32),
            pltpu.VMEM((_NBUF, _B, _CK), jnp.float32),
            pltpu.VMEM_SHARED((_N, _CK), jnp.float32),
            pltpu.SemaphoreType.DMA((_NBUF,)),
            pltpu.SemaphoreType.DMA((_NBUF,)),
        ],
        compiler_params=pltpu.CompilerParams(use_tc_tiling_on_sc=False),
    )


def _chunk_specs(n):
    return [pl.BlockSpec((_R, _CK), lambda i: (i, 0))] * n


def _dinv_of(deg0_ref, deg1_ref):
    return jax.lax.rsqrt(deg0_ref[:, 0] + deg1_ref[:, 0] + 1.0)


def _tc1a_body(x_ref, w1_ref, *xw_refs):
    xw = jnp.dot(x_ref[...], w1_ref[...], preferred_element_type=jnp.float32)
    for k in range(4):
        xw_refs[k][...] = xw[:, k * _CK:(k + 1) * _CK]


def _tc1a(x, w1):
    return pl.pallas_call(
        _tc1a_body,
        grid=(_RB,),
        in_specs=[
            pl.BlockSpec((_R, _IN), lambda i: (i, 0)),
            pl.BlockSpec((_IN, _HID), lambda i: (0, 0)),
        ],
        out_specs=_chunk_specs(4),
        out_shape=[jax.ShapeDtypeStruct((_N, _CK), jnp.float32)] * 4,
    )(x, w1)


def _tc1b_body(deg0_ref, deg1_ref, *refs):
    xw_refs, y_refs = refs[0:4], refs[4:8]
    dinv = _dinv_of(deg0_ref, deg1_ref)
    for k in range(4):
        y_refs[k][...] = dinv[:, None] * xw_refs[k][...]


def _tc1b(deg0, deg1, xws):
    return pl.pallas_call(
        _tc1b_body,
        grid=(_RB,),
        in_specs=[
            pl.BlockSpec((_R, 16), lambda i: (i, 0)),
            pl.BlockSpec((_R, 16), lambda i: (i, 0)),
            *_chunk_specs(4),
        ],
        out_specs=_chunk_specs(4),
        out_shape=[jax.ShapeDtypeStruct((_N, _CK), jnp.float32)] * 4,
    )(deg0, deg1, *xws)


def _sum_rows(a_refs, y_refs):
    return jnp.concatenate(
        [a_refs[k][...] + y_refs[k][...] for k in range(4)], axis=1)


def _tc2_body(deg0_ref, deg1_ref, *refs):
    a_refs, y_refs = refs[0:4], refs[4:8]
    b1_ref, w2_ref = refs[8], refs[9]
    o_refs = refs[10:14]
    dinv = _dinv_of(deg0_ref, deg1_ref)
    h = jnp.maximum(dinv[:, None] * _sum_rows(a_refs, y_refs) + b1_ref[...],
                    0.0)
    y2 = dinv[:, None] * jnp.dot(h, w2_ref[...],
                                 preferred_element_type=jnp.float32)
    for k in range(4):
        o_refs[k][...] = y2[:, k * _CK:(k + 1) * _CK]


def _tc2(deg0, deg1, accs, ys, b1, w2):
    return pl.pallas_call(
        _tc2_body,
        grid=(_RB,),
        in_specs=[
            pl.BlockSpec((_R, 16), lambda i: (i, 0)),
            pl.BlockSpec((_R, 16), lambda i: (i, 0)),
            *_chunk_specs(8),
            pl.BlockSpec((1, _HID), lambda i: (0, 0)),
            pl.BlockSpec((_HID, _HID), lambda i: (0, 0)),
        ],
        out_specs=_chunk_specs(4),
        out_shape=[jax.ShapeDtypeStruct((_N, _CK), jnp.float32)] * 4,
    )(deg0, deg1, *accs, *ys, b1, w2)


def _tc3_body(deg0_ref, deg1_ref, *refs):
    a_refs, y_refs = refs[0:4], refs[4:8]
    b2_ref, batch_ref, wl_ref, bl_ref = refs[8:12]
    out_ref = refs[12]
    pool_acc, cnt_acc = refs[13], refs[14]
    i = pl.program_id(0)
    dinv = _dinv_of(deg0_ref, deg1_ref)
    h = jnp.maximum(dinv[:, None] * _sum_rows(a_refs, y_refs) + b2_ref[...],
                    0.0)
    b = batch_ref[0, 0, :]
    seg = jax.lax.broadcasted_iota(jnp.int32, (_G, _R), 0)
    p = (b[None, :] == seg).astype(jnp.float32)
    ppart = jnp.dot(p, h, preferred_element_type=jnp.float32)
    cpart = jnp.sum(p, axis=1, keepdims=True)

    @pl.when(i == 0)
    def _():
        pool_acc[...] = ppart
        cnt_acc[...] = cpart

    @pl.when(i > 0)
    def _():
        pool_acc[...] += ppart
        cnt_acc[...] += cpart

    @pl.when(i == _RB - 1)
    def _():
        pooled = pool_acc[...] / jnp.maximum(cnt_acc[...], 1.0)
        out_ref[...] = jnp.dot(pooled, wl_ref[...],
                               preferred_element_type=jnp.float32) + bl_ref[...]


def _tc3(deg0, deg1, accs, ys, b2, batch3, wl, bl):
    return pl.pallas_call(
        _tc3_body,
        grid=(_RB,),
        in_specs=[
            pl.BlockSpec((_R, 16), lambda i: (i, 0)),
            pl.BlockSpec((_R, 16), lambda i: (i, 0)),
            *_chunk_specs(8),
            pl.BlockSpec((1, _HID), lambda i: (0, 0)),
            pl.BlockSpec((1, 1, _R), lambda i: (i, 0, 0)),
            pl.BlockSpec((_HID, 1), lambda i: (0, 0)),
            pl.BlockSpec((1, 1), lambda i: (0, 0)),
        ],
        out_specs=pl.BlockSpec((_G, 1), lambda i: (0, 0)),
        out_shape=jax.ShapeDtypeStruct((_G, 1), jnp.float32),
        scratch_shapes=[
            pltpu.VMEM((_G, _HID), jnp.float32),
            pltpu.VMEM((_G, 1), jnp.float32),
        ],
    )(deg0, deg1, *accs, *ys, b2, batch3, wl, bl)


def _deg_call(dst3, zeros):
    return _deg_kernel()(dst3, zeros)


def _layer_call(ys, src3, dst3, zeros):
    return _layer_kernel()(*ys, src3, dst3, zeros)


def kernel(x, edge_index, batch, W1, b1, W2, b2, Wl, bl):
    src = edge_index[0]
    dst = edge_index[1]
    src3 = src.reshape(_NS, _NB, _B)
    dst3 = dst.reshape(_NS, _NB, _B)
    zeros = jnp.zeros((_RPS, _CK), jnp.float32)
    zeros16 = jnp.zeros((_RPS, 16), jnp.float32)
    deg0, deg1 = _deg_call(dst3, zeros16)
    xws = _tc1a(x, W1)
    ys = _tc1b(deg0, deg1, xws)
    accs = _layer_call(ys, src3, dst3, zeros)
    zs = _tc2(deg0, deg1, accs, ys, b1.reshape(1, _HID), W2)
    accs2 = _layer_call(zs, src3, dst3, zeros)
    return _tc3(deg0, deg1, accs2, zs, b2.reshape(1, _HID),
                batch.reshape(_RB, 1, _R), Wl, bl.reshape(1, 1))
